# baseline (speedup 1.0000x reference)
import functools

import jax
import jax.numpy as jnp
from jax import lax
from jax.experimental import pallas as pl
from jax.experimental.pallas import tpu as pltpu
from jax.experimental.pallas import tpu_sc as plsc

N = 2500
E = 10000
G = 128
IN = 32
H1 = 120
H2 = 210
NP = 2560
EP = 10240
CH = 64
CPW = 5
O1 = 128
O2 = 256
EB = 2048
NB = EP // EB


def _sc_gather(table, idx2, d):
    mesh = plsc.VectorSubcoreMesh(core_axis_name="c", subcore_axis_name="s")

    @functools.partial(
        pl.kernel,
        out_type=jax.ShapeDtypeStruct((EP, d), jnp.float32),
        mesh=mesh,
        scratch_types=[
            pltpu.VMEM((CPW, CH), jnp.int32),
            pltpu.VMEM((CPW * CH, d), jnp.float32),
            pltpu.SemaphoreType.DMA,
        ],
    )
    def k(table_hbm, idx_hbm, out_hbm, idx_v, rows_v, sem):
        w = lax.axis_index("s") * 2 + lax.axis_index("c")
        pltpu.sync_copy(idx_hbm.at[w], idx_v)
        descs = [
            pltpu.async_copy(table_hbm.at[idx_v.at[j]],
                             rows_v.at[pl.ds(j * CH, CH)], sem)
            for j in range(CPW)
        ]
        for dsc in descs:
            dsc.wait()
        pltpu.sync_copy(rows_v, out_hbm.at[pl.ds(w * CPW * CH, CPW * CH)])

    return k(table, idx2)


def _zform_msg(ea, feat, wa, ba, wflat, bmat, blk, o, hcol):
    h = jnp.maximum(
        jnp.dot(ea, wa, preferred_element_type=jnp.float32) + ba, 0.0)
    z = jnp.concatenate([h[:, k:k + 1] * feat for k in range(32)], axis=1)
    msg = (jnp.dot(z, wflat, preferred_element_type=jnp.float32)
           + jnp.dot(feat, bmat, preferred_element_type=jnp.float32))
    row = blk * EB + lax.broadcasted_iota(jnp.int32, (EB, o), 0)
    lane = lax.broadcasted_iota(jnp.int32, (EB, o), 1)
    realf = (row < E).astype(jnp.float32)
    return jnp.where(lane == hcol, realf, msg * realf)


def _segsum_step(acc_ref, dst, msg):
    i = pl.program_id(0)

    @pl.when(i == 0)
    def _():
        acc_ref[...] = jnp.zeros_like(acc_ref)

    oh = (lax.broadcasted_iota(jnp.int32, (NP, EB), 0) == dst
          ).astype(jnp.float32)
    acc_ref[...] += jnp.dot(oh, msg, preferred_element_type=jnp.float32)


def _segmean(s, o, hcol):
    sel = (lax.broadcasted_iota(jnp.int32, (o, o), 0) == hcol)
    cnt = jnp.dot(s, sel.astype(jnp.float32),
                  preferred_element_type=jnp.float32)
    lane = lax.broadcasted_iota(jnp.int32, (NP, o), 1)
    return jnp.where(lane < hcol, s, 0.0) / jnp.maximum(cnt, 1.0)


def _bn(h, gam, bet, o):
    rowf = (lax.broadcasted_iota(jnp.int32, (NP, o), 0) < N
            ).astype(jnp.float32)
    m = jnp.sum(h * rowf, axis=0, keepdims=True) * (1.0 / N)
    d = (h - m) * rowf
    v = jnp.sum(d * d, axis=0, keepdims=True) * (1.0 / N)
    return (h - m) * lax.rsqrt(v + 1e-5) * gam + bet


def _layer1_body(dst_ref, ea_ref, feat_ref, wa_ref, ba_ref, wflat_ref,
                 bmat_ref, x_ref, root_ref, bias_ref, gam_ref, bet_ref,
                 out_ref, acc_ref):
    i = pl.program_id(0)
    h = jnp.maximum(
        jnp.dot(ea_ref[...], wa_ref[...],
                preferred_element_type=jnp.float32) + ba_ref[...], 0.0)
    feat = feat_ref[:, :32]
    sel = (lax.broadcasted_iota(jnp.int32, (32, 1024), 0)
           == lax.broadcasted_iota(jnp.int32, (32, 1024), 1) // 32
           ).astype(jnp.float32)
    hrep = jnp.dot(h, sel, preferred_element_type=jnp.float32)
    ztile = jnp.tile(jnp.tile(feat, (1, 4)), (1, 8))
    z = hrep * ztile
    msg = (jnp.dot(z, wflat_ref[...], preferred_element_type=jnp.float32)
           + jnp.dot(feat, bmat_ref[...], preferred_element_type=jnp.float32))
    row = i * EB + lax.broadcasted_iota(jnp.int32, (EB, O1), 0)
    lane = lax.broadcasted_iota(jnp.int32, (EB, O1), 1)
    realf = (row < E).astype(jnp.float32)
    msg = jnp.where(lane == H1, realf, msg * realf)
    _segsum_step(acc_ref, dst_ref[...], msg)

    @pl.when(i == NB - 1)
    def _():
        agg = _segmean(acc_ref[...], O1, H1)
        h = jnp.maximum(
            agg + jnp.dot(x_ref[:, :32], root_ref[...],
                          preferred_element_type=jnp.float32)
            + bias_ref[...], 0.0)
        out_ref[...] = _bn(h, gam_ref[...], bet_ref[...], O1)


def _layer2_body(dst_ref, ea_ref, feat_ref, wa_ref, ba_ref, wflat_ref,
                 bmat_ref, h1_ref, root_ref, bias_ref, gam_ref, bet_ref,
                 batch_ref, w3_ref, b3_ref, w4_ref, b4_ref,
                 out_ref, acc_ref):
    i = pl.program_id(0)
    msg = _zform_msg(ea_ref[...], feat_ref[...], wa_ref[...], ba_ref[...],
                     wflat_ref[...], bmat_ref[...], i, O2, H2)
    _segsum_step(acc_ref, dst_ref[...], msg)

    @pl.when(i == NB - 1)
    def _():
        agg = _segmean(acc_ref[...], O2, H2)
        h = jnp.maximum(
            agg + jnp.dot(h1_ref[...], root_ref[...],
                          preferred_element_type=jnp.float32)
            + bias_ref[...], 0.0)
        hbn = _bn(h, gam_ref[...], bet_ref[...], O2)
        oh = (lax.broadcasted_iota(jnp.int32, (G, NP), 0)
              == batch_ref[...]).astype(jnp.float32)
        gs = jnp.dot(oh, hbn, preferred_element_type=jnp.float32)
        gc = jnp.sum(oh, axis=1, keepdims=True)
        g = gs / jnp.maximum(gc, 1.0)
        g = jnp.maximum(
            jnp.dot(g, w3_ref[...], preferred_element_type=jnp.float32)
            + b3_ref[...], 0.0)
        out_ref[...] = (jnp.dot(g, w4_ref[...],
                                preferred_element_type=jnp.float32)
                        + b4_ref[...])


def _tc_layer1(dst_row, ea, xs, wa, ba, wflat, bmat, xp, rootp, biasp,
               gamp, betp):
    return pl.pallas_call(
        _layer1_body,
        grid=(NB,),
        in_specs=[
            pl.BlockSpec((1, EB), lambda i: (0, i)),
            pl.BlockSpec((EB, 16), lambda i: (i, 0)),
            pl.BlockSpec((EB, 128), lambda i: (i, 0)),
            pl.BlockSpec((16, 32), lambda i: (0, 0)),
            pl.BlockSpec((1, 32), lambda i: (0, 0)),
            pl.BlockSpec((1024, O1), lambda i: (0, 0)),
            pl.BlockSpec((32, O1), lambda i: (0, 0)),
            pl.BlockSpec((NP, 128), lambda i: (0, 0)),
            pl.BlockSpec((32, O1), lambda i: (0, 0)),
            pl.BlockSpec((1, O1), lambda i: (0, 0)),
            pl.BlockSpec((1, O1), lambda i: (0, 0)),
            pl.BlockSpec((1, O1), lambda i: (0, 0)),
        ],
        out_specs=pl.BlockSpec((NP, O1), lambda i: (0, 0)),
        out_shape=jax.ShapeDtypeStruct((NP, O1), jnp.float32),
        scratch_shapes=[pltpu.VMEM((NP, O1), jnp.float32)],
    )(dst_row, ea, xs, wa, ba, wflat, bmat, xp, rootp, biasp, gamp, betp)


def _tc_layer2(dst_row, ea, hs, wa, ba, wflat, bmat, h1bn, rootp, biasp,
               gamp, betp, batch_row, w3p, b3p, w4p, b4b):
    return pl.pallas_call(
        _layer2_body,
        grid=(NB,),
        in_specs=[
            pl.BlockSpec((1, EB), lambda i: (0, i)),
            pl.BlockSpec((EB, 16), lambda i: (i, 0)),
            pl.BlockSpec((EB, 128), lambda i: (i, 0)),
            pl.BlockSpec((16, 32), lambda i: (0, 0)),
            pl.BlockSpec((1, 32), lambda i: (0, 0)),
            pl.BlockSpec((4096, O2), lambda i: (0, 0)),
            pl.BlockSpec((128, O2), lambda i: (0, 0)),
            pl.BlockSpec((NP, 128), lambda i: (0, 0)),
            pl.BlockSpec((128, O2), lambda i: (0, 0)),
            pl.BlockSpec((1, O2), lambda i: (0, 0)),
            pl.BlockSpec((1, O2), lambda i: (0, 0)),
            pl.BlockSpec((1, O2), lambda i: (0, 0)),
            pl.BlockSpec((1, NP), lambda i: (0, 0)),
            pl.BlockSpec((O2, 128), lambda i: (0, 0)),
            pl.BlockSpec((1, 128), lambda i: (0, 0)),
            pl.BlockSpec((128, 128), lambda i: (0, 0)),
            pl.BlockSpec((1, 128), lambda i: (0, 0)),
        ],
        out_specs=pl.BlockSpec((G, 128), lambda i: (0, 0)),
        out_shape=jax.ShapeDtypeStruct((G, 128), jnp.float32),
        scratch_shapes=[pltpu.VMEM((NP, O2), jnp.float32)],
    )(dst_row, ea, hs, wa, ba, wflat, bmat, h1bn, rootp, biasp, gamp, betp,
      batch_row, w3p, b3p, w4p, b4b)


def _pad2(a, r, c):
    return jnp.pad(a, ((0, r - a.shape[0]), (0, c - a.shape[1])))


def kernel(x, edge_index, edge_attr, batch, W1a, b1a, W1b, b1b, root1, bias1,
           gamma1, beta1, W2a, b2a, W2b, b2b, root2, bias2, gamma2, beta2,
           W3, b3, W4, b4):
    f32 = jnp.float32

    src2 = jnp.pad(edge_index[0], (0, EP - E)).reshape(32, CPW, CH)
    dst_row = jnp.pad(edge_index[1], (0, EP - E)).reshape(1, EP)
    ea_p = jnp.pad(edge_attr, ((0, EP - E), (0, 0)))
    x_p = _pad2(x, NP, 128)
    batch_row = jnp.pad(batch, (0, NP - N), constant_values=-1).reshape(1, NP)

    w1flat = jnp.pad(W1b.reshape(32, IN, H1),
                     ((0, 0), (0, 0), (0, O1 - H1))).reshape(32 * IN, O1)
    b1mat = _pad2(b1b.reshape(IN, H1), IN, O1)
    w2flat = jnp.pad(W2b.reshape(32, H1, H2),
                     ((0, 0), (0, 128 - H1), (0, O2 - H2))).reshape(32 * 128, O2)
    b2mat = _pad2(b2b.reshape(H1, H2), 128, O2)

    root1p = _pad2(root1, 32, O1)
    root2p = _pad2(root2, 128, O2)
    bias1p = jnp.pad(bias1, (0, O1 - H1)).reshape(1, O1)
    gam1p = jnp.pad(gamma1, (0, O1 - H1)).reshape(1, O1)
    bet1p = jnp.pad(beta1, (0, O1 - H1)).reshape(1, O1)
    bias2p = jnp.pad(bias2, (0, O2 - H2)).reshape(1, O2)
    gam2p = jnp.pad(gamma2, (0, O2 - H2)).reshape(1, O2)
    bet2p = jnp.pad(beta2, (0, O2 - H2)).reshape(1, O2)
    w3p = _pad2(W3, O2, 128)
    b3p = jnp.pad(b3, (0, 128 - 64)).reshape(1, 128)
    w4p = _pad2(W4, 128, 128)
    b4b = jnp.broadcast_to(b4.reshape(1, 1), (1, 128))
    ba1 = b1a.reshape(1, 32)
    ba2 = b2a.reshape(1, 32)

    xs = _sc_gather(x_p, src2, 128)
    h1bn = _tc_layer1(dst_row, ea_p, xs, W1a, ba1, w1flat, b1mat,
                      x_p, root1p, bias1p, gam1p, bet1p)

    hs = _sc_gather(h1bn, src2, 128)
    out = _tc_layer2(dst_row, ea_p, hs, W2a, ba2, w2flat, b2mat,
                     h1bn, root2p, bias2p, gam2p, bet2p, batch_row,
                     w3p, b3p, w4p, b4b)
    return out[:, 0]


# speedup vs baseline: 1.0100x; 1.0100x over previous
import functools

import jax
import jax.numpy as jnp
from jax import lax
from jax.experimental import pallas as pl
from jax.experimental.pallas import tpu as pltpu
from jax.experimental.pallas import tpu_sc as plsc

N = 2500
E = 10000
G = 128
IN = 32
H1 = 120
H2 = 210
NP = 2560
EP = 10240
CH = 64
CPW = 5
O1 = 128
O2 = 256
EB = 1024
NB = EP // EB


def _sc_gather(table, idx2, d):
    mesh = plsc.VectorSubcoreMesh(core_axis_name="c", subcore_axis_name="s")

    @functools.partial(
        pl.kernel,
        out_type=jax.ShapeDtypeStruct((EP, d), jnp.float32),
        mesh=mesh,
        scratch_types=[
            pltpu.VMEM((CPW, CH), jnp.int32),
            pltpu.VMEM((CPW * CH, d), jnp.float32),
            pltpu.SemaphoreType.DMA,
            pltpu.SemaphoreType.DMA,
        ],
    )
    def k(table_hbm, idx_hbm, out_hbm, idx_v, rows_v, sem, wsem):
        w = lax.axis_index("s") * 2 + lax.axis_index("c")
        pltpu.sync_copy(idx_hbm.at[w], idx_v)
        descs = [
            pltpu.async_copy(table_hbm.at[idx_v.at[j]],
                             rows_v.at[pl.ds(j * CH, CH)], sem)
            for j in range(CPW)
        ]
        wdescs = []
        for j, dsc in enumerate(descs):
            dsc.wait()
            wdescs.append(
                pltpu.async_copy(rows_v.at[pl.ds(j * CH, CH)],
                                 out_hbm.at[pl.ds((w * CPW + j) * CH, CH)],
                                 wsem))
        for dsc in wdescs:
            dsc.wait()

    return k(table, idx2)


def _zform_msg(ea, feat, wa, ba, wflat, bmat, blk, o, hcol):
    h = jnp.maximum(
        jnp.dot(ea, wa, preferred_element_type=jnp.float32) + ba, 0.0)
    z = jnp.concatenate([h[:, k:k + 1] * feat for k in range(32)], axis=1)
    msg = (jnp.dot(z, wflat, preferred_element_type=jnp.float32)
           + jnp.dot(feat, bmat, preferred_element_type=jnp.float32))
    row = blk * EB + lax.broadcasted_iota(jnp.int32, (EB, o), 0)
    lane = lax.broadcasted_iota(jnp.int32, (EB, o), 1)
    realf = (row < E).astype(jnp.float32)
    return jnp.where(lane == hcol, realf, msg * realf)


def _segsum_step(acc_ref, dst, msg):
    i = pl.program_id(0)

    @pl.when(i == 0)
    def _():
        acc_ref[...] = jnp.zeros_like(acc_ref)

    oh = (lax.broadcasted_iota(jnp.int32, (NP, EB), 0) == dst
          ).astype(jnp.float32)
    acc_ref[...] += jnp.dot(oh, msg, preferred_element_type=jnp.float32)


def _segmean(s, o, hcol):
    sel = (lax.broadcasted_iota(jnp.int32, (o, o), 0) == hcol)
    cnt = jnp.dot(s, sel.astype(jnp.float32),
                  preferred_element_type=jnp.float32)
    lane = lax.broadcasted_iota(jnp.int32, (NP, o), 1)
    return jnp.where(lane < hcol, s, 0.0) / jnp.maximum(cnt, 1.0)


def _bn(h, gam, bet, o):
    rowf = (lax.broadcasted_iota(jnp.int32, (NP, o), 0) < N
            ).astype(jnp.float32)
    m = jnp.sum(h * rowf, axis=0, keepdims=True) * (1.0 / N)
    d = (h - m) * rowf
    v = jnp.sum(d * d, axis=0, keepdims=True) * (1.0 / N)
    return (h - m) * lax.rsqrt(v + 1e-5) * gam + bet


def _layer1_body(dst_ref, ea_ref, feat_ref, wa_ref, ba_ref, wflat_ref,
                 bmat_ref, x_ref, root_ref, bias_ref, gam_ref, bet_ref,
                 out_ref, acc_ref):
    i = pl.program_id(0)
    h = jnp.maximum(
        jnp.dot(ea_ref[...], wa_ref[...],
                preferred_element_type=jnp.float32) + ba_ref[...], 0.0)
    feat = feat_ref[:, :32]
    sel = (lax.broadcasted_iota(jnp.int32, (32, 1024), 0)
           == lax.broadcasted_iota(jnp.int32, (32, 1024), 1) // 32
           ).astype(jnp.float32)
    hrep = jnp.dot(h, sel, preferred_element_type=jnp.float32)
    ztile = jnp.tile(jnp.tile(feat, (1, 4)), (1, 8))
    z = hrep * ztile
    msg = (jnp.dot(z, wflat_ref[...], preferred_element_type=jnp.float32)
           + jnp.dot(feat, bmat_ref[...], preferred_element_type=jnp.float32))
    row = i * EB + lax.broadcasted_iota(jnp.int32, (EB, O1), 0)
    lane = lax.broadcasted_iota(jnp.int32, (EB, O1), 1)
    realf = (row < E).astype(jnp.float32)
    msg = jnp.where(lane == H1, realf, msg * realf)
    _segsum_step(acc_ref, dst_ref[...], msg)

    @pl.when(i == NB - 1)
    def _():
        agg = _segmean(acc_ref[...], O1, H1)
        h = jnp.maximum(
            agg + jnp.dot(x_ref[:, :32], root_ref[...],
                          preferred_element_type=jnp.float32)
            + bias_ref[...], 0.0)
        out_ref[...] = _bn(h, gam_ref[...], bet_ref[...], O1)


def _layer2_body(dst_ref, ea_ref, feat_ref, wa_ref, ba_ref, wflat_ref,
                 bmat_ref, h1_ref, root_ref, bias_ref, gam_ref, bet_ref,
                 batch_ref, w3_ref, b3_ref, w4_ref, b4_ref,
                 out_ref, acc_ref):
    i = pl.program_id(0)
    msg = _zform_msg(ea_ref[...], feat_ref[...], wa_ref[...], ba_ref[...],
                     wflat_ref[...], bmat_ref[...], i, O2, H2)
    _segsum_step(acc_ref, dst_ref[...], msg)

    @pl.when(i == NB - 1)
    def _():
        agg = _segmean(acc_ref[...], O2, H2)
        h = jnp.maximum(
            agg + jnp.dot(h1_ref[...], root_ref[...],
                          preferred_element_type=jnp.float32)
            + bias_ref[...], 0.0)
        hbn = _bn(h, gam_ref[...], bet_ref[...], O2)
        oh = (lax.broadcasted_iota(jnp.int32, (G, NP), 0)
              == batch_ref[...]).astype(jnp.float32)
        gs = jnp.dot(oh, hbn, preferred_element_type=jnp.float32)
        gc = jnp.sum(oh, axis=1, keepdims=True)
        g = gs / jnp.maximum(gc, 1.0)
        g = jnp.maximum(
            jnp.dot(g, w3_ref[...], preferred_element_type=jnp.float32)
            + b3_ref[...], 0.0)
        out_ref[...] = (jnp.dot(g, w4_ref[...],
                                preferred_element_type=jnp.float32)
                        + b4_ref[...])


def _tc_layer1(dst_row, ea, xs, wa, ba, wflat, bmat, xp, rootp, biasp,
               gamp, betp):
    return pl.pallas_call(
        _layer1_body,
        grid=(NB,),
        in_specs=[
            pl.BlockSpec((1, EB), lambda i: (0, i)),
            pl.BlockSpec((EB, 16), lambda i: (i, 0)),
            pl.BlockSpec((EB, 128), lambda i: (i, 0)),
            pl.BlockSpec((16, 32), lambda i: (0, 0)),
            pl.BlockSpec((1, 32), lambda i: (0, 0)),
            pl.BlockSpec((1024, O1), lambda i: (0, 0)),
            pl.BlockSpec((32, O1), lambda i: (0, 0)),
            pl.BlockSpec((NP, 128), lambda i: (0, 0)),
            pl.BlockSpec((32, O1), lambda i: (0, 0)),
            pl.BlockSpec((1, O1), lambda i: (0, 0)),
            pl.BlockSpec((1, O1), lambda i: (0, 0)),
            pl.BlockSpec((1, O1), lambda i: (0, 0)),
        ],
        out_specs=pl.BlockSpec((NP, O1), lambda i: (0, 0)),
        out_shape=jax.ShapeDtypeStruct((NP, O1), jnp.float32),
        scratch_shapes=[pltpu.VMEM((NP, O1), jnp.float32)],
    )(dst_row, ea, xs, wa, ba, wflat, bmat, xp, rootp, biasp, gamp, betp)


def _tc_layer2(dst_row, ea, hs, wa, ba, wflat, bmat, h1bn, rootp, biasp,
               gamp, betp, batch_row, w3p, b3p, w4p, b4b):
    return pl.pallas_call(
        _layer2_body,
        grid=(NB,),
        in_specs=[
            pl.BlockSpec((1, EB), lambda i: (0, i)),
            pl.BlockSpec((EB, 16), lambda i: (i, 0)),
            pl.BlockSpec((EB, 128), lambda i: (i, 0)),
            pl.BlockSpec((16, 32), lambda i: (0, 0)),
            pl.BlockSpec((1, 32), lambda i: (0, 0)),
            pl.BlockSpec((4096, O2), lambda i: (0, 0)),
            pl.BlockSpec((128, O2), lambda i: (0, 0)),
            pl.BlockSpec((NP, 128), lambda i: (0, 0)),
            pl.BlockSpec((128, O2), lambda i: (0, 0)),
            pl.BlockSpec((1, O2), lambda i: (0, 0)),
            pl.BlockSpec((1, O2), lambda i: (0, 0)),
            pl.BlockSpec((1, O2), lambda i: (0, 0)),
            pl.BlockSpec((1, NP), lambda i: (0, 0)),
            pl.BlockSpec((O2, 128), lambda i: (0, 0)),
            pl.BlockSpec((1, 128), lambda i: (0, 0)),
            pl.BlockSpec((128, 128), lambda i: (0, 0)),
            pl.BlockSpec((1, 128), lambda i: (0, 0)),
        ],
        out_specs=pl.BlockSpec((G, 128), lambda i: (0, 0)),
        out_shape=jax.ShapeDtypeStruct((G, 128), jnp.float32),
        scratch_shapes=[pltpu.VMEM((NP, O2), jnp.float32)],
    )(dst_row, ea, hs, wa, ba, wflat, bmat, h1bn, rootp, biasp, gamp, betp,
      batch_row, w3p, b3p, w4p, b4b)


def _pad2(a, r, c):
    return jnp.pad(a, ((0, r - a.shape[0]), (0, c - a.shape[1])))


def kernel(x, edge_index, edge_attr, batch, W1a, b1a, W1b, b1b, root1, bias1,
           gamma1, beta1, W2a, b2a, W2b, b2b, root2, bias2, gamma2, beta2,
           W3, b3, W4, b4):
    f32 = jnp.float32

    src2 = jnp.pad(edge_index[0], (0, EP - E)).reshape(32, CPW, CH)
    dst_row = jnp.pad(edge_index[1], (0, EP - E)).reshape(1, EP)
    ea_p = jnp.pad(edge_attr, ((0, EP - E), (0, 0)))
    x_p = _pad2(x, NP, 128)
    batch_row = jnp.pad(batch, (0, NP - N), constant_values=-1).reshape(1, NP)

    w1flat = jnp.pad(W1b.reshape(32, IN, H1),
                     ((0, 0), (0, 0), (0, O1 - H1))).reshape(32 * IN, O1)
    b1mat = _pad2(b1b.reshape(IN, H1), IN, O1)
    w2flat = jnp.pad(W2b.reshape(32, H1, H2),
                     ((0, 0), (0, 128 - H1), (0, O2 - H2))).reshape(32 * 128, O2)
    b2mat = _pad2(b2b.reshape(H1, H2), 128, O2)

    root1p = _pad2(root1, 32, O1)
    root2p = _pad2(root2, 128, O2)
    bias1p = jnp.pad(bias1, (0, O1 - H1)).reshape(1, O1)
    gam1p = jnp.pad(gamma1, (0, O1 - H1)).reshape(1, O1)
    bet1p = jnp.pad(beta1, (0, O1 - H1)).reshape(1, O1)
    bias2p = jnp.pad(bias2, (0, O2 - H2)).reshape(1, O2)
    gam2p = jnp.pad(gamma2, (0, O2 - H2)).reshape(1, O2)
    bet2p = jnp.pad(beta2, (0, O2 - H2)).reshape(1, O2)
    w3p = _pad2(W3, O2, 128)
    b3p = jnp.pad(b3, (0, 128 - 64)).reshape(1, 128)
    w4p = _pad2(W4, 128, 128)
    b4b = jnp.broadcast_to(b4.reshape(1, 1), (1, 128))
    ba1 = b1a.reshape(1, 32)
    ba2 = b2a.reshape(1, 32)

    xs = _sc_gather(x_p, src2, 128)
    h1bn = _tc_layer1(dst_row, ea_p, xs, W1a, ba1, w1flat, b1mat,
                      x_p, root1p, bias1p, gam1p, bet1p)

    hs = _sc_gather(h1bn, src2, 128)
    out = _tc_layer2(dst_row, ea_p, hs, W2a, ba2, w2flat, b2mat,
                     h1bn, root2p, bias2p, gam2p, bet2p, batch_row,
                     w3p, b3p, w4p, b4b)
    return out[:, 0]


# speedup vs baseline: 1.0122x; 1.0022x over previous
import functools

import jax
import jax.numpy as jnp
from jax import lax
from jax.experimental import pallas as pl
from jax.experimental.pallas import tpu as pltpu
from jax.experimental.pallas import tpu_sc as plsc

N = 2500
E = 10000
G = 128
IN = 32
H1 = 120
H2 = 210
NP = 2560
EP = 10240
CH = 64
CPW = 5
O1 = 128
O2 = 256
EB = 1024
NB = EP // EB


def _sc_gather(table, idx2, d):
    mesh = plsc.VectorSubcoreMesh(core_axis_name="c", subcore_axis_name="s")

    @functools.partial(
        pl.kernel,
        out_type=jax.ShapeDtypeStruct((EP, d), jnp.float32),
        mesh=mesh,
        scratch_types=[
            pltpu.VMEM((CPW, CH), jnp.int32),
            pltpu.VMEM((CPW * CH, d), jnp.float32),
            pltpu.SemaphoreType.DMA,
            pltpu.SemaphoreType.DMA,
        ],
    )
    def k(table_hbm, idx_hbm, out_hbm, idx_v, rows_v, sem, wsem):
        w = lax.axis_index("s") * 2 + lax.axis_index("c")
        pltpu.sync_copy(idx_hbm.at[w], idx_v)
        descs = [
            pltpu.async_copy(table_hbm.at[idx_v.at[j]],
                             rows_v.at[pl.ds(j * CH, CH)], sem)
            for j in range(CPW)
        ]
        wdescs = []
        for j, dsc in enumerate(descs):
            dsc.wait()
            wdescs.append(
                pltpu.async_copy(rows_v.at[pl.ds(j * CH, CH)],
                                 out_hbm.at[pl.ds((w * CPW + j) * CH, CH)],
                                 wsem))
        for dsc in wdescs:
            dsc.wait()

    return k(table, idx2)


def _zform_msg(ea, feat, wa, ba, wflat, bmat, blk, o, hcol):
    h = jnp.maximum(
        jnp.dot(ea, wa, preferred_element_type=jnp.float32) + ba, 0.0)
    z = jnp.concatenate([h[:, k:k + 1] * feat for k in range(32)], axis=1)
    msg = (jnp.dot(z, wflat, preferred_element_type=jnp.float32)
           + jnp.dot(feat, bmat, preferred_element_type=jnp.float32))
    row = blk * EB + lax.broadcasted_iota(jnp.int32, (EB, o), 0)
    lane = lax.broadcasted_iota(jnp.int32, (EB, o), 1)
    realf = (row < E).astype(jnp.float32)
    return jnp.where(lane == hcol, realf, msg * realf)


def _segsum_step(acc_ref, dst, msg):
    i = pl.program_id(0)

    @pl.when(i == 0)
    def _():
        acc_ref[...] = jnp.zeros_like(acc_ref)

    oh = (lax.broadcasted_iota(jnp.int32, (NP, EB), 0) == dst
          ).astype(jnp.float32)
    acc_ref[...] += jnp.dot(oh, msg, preferred_element_type=jnp.float32)


def _segmean(s, o, hcol):
    sel = (lax.broadcasted_iota(jnp.int32, (o, o), 0) == hcol)
    cnt = jnp.dot(s, sel.astype(jnp.float32),
                  preferred_element_type=jnp.float32)
    lane = lax.broadcasted_iota(jnp.int32, (NP, o), 1)
    return jnp.where(lane < hcol, s, 0.0) / jnp.maximum(cnt, 1.0)


def _bn(h, gam, bet, o):
    rowf = (lax.broadcasted_iota(jnp.int32, (NP, o), 0) < N
            ).astype(jnp.float32)
    m = jnp.sum(h * rowf, axis=0, keepdims=True) * (1.0 / N)
    d = (h - m) * rowf
    v = jnp.sum(d * d, axis=0, keepdims=True) * (1.0 / N)
    return (h - m) * lax.rsqrt(v + 1e-5) * gam + bet


def _layer1_body(dst_ref, ea_ref, feat_ref, wa_ref, ba_ref, wflat_ref,
                 bmat_ref, x_ref, root_ref, bias_ref, gam_ref, bet_ref,
                 out_ref, acc_ref):
    i = pl.program_id(0)
    h = jnp.maximum(
        jnp.dot(ea_ref[...], wa_ref[...],
                preferred_element_type=jnp.float32) + ba_ref[...], 0.0)
    feat = feat_ref[:, :32]
    sel = (lax.broadcasted_iota(jnp.int32, (32, 1024), 0)
           == lax.broadcasted_iota(jnp.int32, (32, 1024), 1) // 32
           ).astype(jnp.float32)
    hrep = jnp.dot(h, sel, preferred_element_type=jnp.float32)
    ztile = jnp.tile(jnp.tile(feat, (1, 4)), (1, 8))
    z = hrep * ztile
    msg = (jnp.dot(z, wflat_ref[...], preferred_element_type=jnp.float32)
           + jnp.dot(feat, bmat_ref[...], preferred_element_type=jnp.float32))
    row = i * EB + lax.broadcasted_iota(jnp.int32, (EB, O1), 0)
    lane = lax.broadcasted_iota(jnp.int32, (EB, O1), 1)
    realf = (row < E).astype(jnp.float32)
    msg = jnp.where(lane == H1, realf, msg * realf)
    _segsum_step(acc_ref, dst_ref[...], msg)

    @pl.when(i == NB - 1)
    def _():
        agg = _segmean(acc_ref[...], O1, H1)
        h = jnp.maximum(
            agg + jnp.dot(x_ref[:, :32], root_ref[...],
                          preferred_element_type=jnp.float32)
            + bias_ref[...], 0.0)
        out_ref[...] = _bn(h, gam_ref[...], bet_ref[...], O1)


def _layer2_body(dst_ref, ea_ref, feat_ref, wa_ref, ba_ref, wflat_ref,
                 bmat_ref, h1_ref, root_ref, bias_ref, gam_ref, bet_ref,
                 batch_ref, w3_ref, b3_ref, w4_ref, b4_ref,
                 out_ref, acc_ref):
    i = pl.program_id(0)
    msg = _zform_msg(ea_ref[...], feat_ref[...], wa_ref[...], ba_ref[...],
                     wflat_ref[...], bmat_ref[...], i, O2, H2)
    _segsum_step(acc_ref, dst_ref[...], msg)

    @pl.when(i == NB - 1)
    def _():
        agg = _segmean(acc_ref[...], O2, H2)
        h = jnp.maximum(
            agg + jnp.dot(h1_ref[...], root_ref[...],
                          preferred_element_type=jnp.float32)
            + bias_ref[...], 0.0)
        hbn = _bn(h, gam_ref[...], bet_ref[...], O2)
        oh = (lax.broadcasted_iota(jnp.int32, (G, NP), 0)
              == batch_ref[...]).astype(jnp.float32)
        gs = jnp.dot(oh, hbn, preferred_element_type=jnp.float32)
        gc = jnp.sum(oh, axis=1, keepdims=True)
        g = gs / jnp.maximum(gc, 1.0)
        g = jnp.maximum(
            jnp.dot(g, w3_ref[...], preferred_element_type=jnp.float32)
            + b3_ref[...], 0.0)
        out_ref[...] = (jnp.dot(g, w4_ref[...],
                                preferred_element_type=jnp.float32)
                        + b4_ref[...])


def _tc_layer1(dst_row, ea, xs, wa, ba, wflat, bmat, xp, rootp, biasp,
               gamp, betp):
    return pl.pallas_call(
        _layer1_body,
        grid=(NB,),
        in_specs=[
            pl.BlockSpec((1, EB), lambda i: (0, i)),
            pl.BlockSpec((EB, 16), lambda i: (i, 0)),
            pl.BlockSpec((EB, 128), lambda i: (i, 0)),
            pl.BlockSpec((16, 32), lambda i: (0, 0)),
            pl.BlockSpec((1, 32), lambda i: (0, 0)),
            pl.BlockSpec((1024, O1), lambda i: (0, 0)),
            pl.BlockSpec((32, O1), lambda i: (0, 0)),
            pl.BlockSpec((NP, 128), lambda i: (0, 0)),
            pl.BlockSpec((32, O1), lambda i: (0, 0)),
            pl.BlockSpec((1, O1), lambda i: (0, 0)),
            pl.BlockSpec((1, O1), lambda i: (0, 0)),
            pl.BlockSpec((1, O1), lambda i: (0, 0)),
        ],
        out_specs=pl.BlockSpec((NP, O1), lambda i: (0, 0)),
        out_shape=jax.ShapeDtypeStruct((NP, O1), jnp.float32),
        scratch_shapes=[pltpu.VMEM((NP, O1), jnp.float32)],
    )(dst_row, ea, xs, wa, ba, wflat, bmat, xp, rootp, biasp, gamp, betp)


def _tc_layer2(dst_row, ea, hs, wa, ba, wflat, bmat, h1bn, rootp, biasp,
               gamp, betp, batch_row, w3p, b3p, w4p, b4b):
    return pl.pallas_call(
        _layer2_body,
        grid=(NB,),
        in_specs=[
            pl.BlockSpec((1, EB), lambda i: (0, i)),
            pl.BlockSpec((EB, 16), lambda i: (i, 0)),
            pl.BlockSpec((EB, 128), lambda i: (i, 0)),
            pl.BlockSpec((16, 32), lambda i: (0, 0)),
            pl.BlockSpec((1, 32), lambda i: (0, 0)),
            pl.BlockSpec((4096, O2), lambda i: (0, 0)),
            pl.BlockSpec((128, O2), lambda i: (0, 0)),
            pl.BlockSpec((NP, 128), lambda i: (0, 0)),
            pl.BlockSpec((128, O2), lambda i: (0, 0)),
            pl.BlockSpec((1, O2), lambda i: (0, 0)),
            pl.BlockSpec((1, O2), lambda i: (0, 0)),
            pl.BlockSpec((1, O2), lambda i: (0, 0)),
            pl.BlockSpec((1, NP), lambda i: (0, 0)),
            pl.BlockSpec((O2, 128), lambda i: (0, 0)),
            pl.BlockSpec((1, 128), lambda i: (0, 0)),
            pl.BlockSpec((128, 128), lambda i: (0, 0)),
            pl.BlockSpec((1, 128), lambda i: (0, 0)),
        ],
        out_specs=pl.BlockSpec((G, 128), lambda i: (0, 0)),
        out_shape=jax.ShapeDtypeStruct((G, 128), jnp.float32),
        scratch_shapes=[pltpu.VMEM((NP, O2), jnp.float32)],
    )(dst_row, ea, hs, wa, ba, wflat, bmat, h1bn, rootp, biasp, gamp, betp,
      batch_row, w3p, b3p, w4p, b4b)


def _pad2(a, r, c):
    return jnp.pad(a, ((0, r - a.shape[0]), (0, c - a.shape[1])))


def kernel(x, edge_index, edge_attr, batch, W1a, b1a, W1b, b1b, root1, bias1,
           gamma1, beta1, W2a, b2a, W2b, b2b, root2, bias2, gamma2, beta2,
           W3, b3, W4, b4):
    src2 = jnp.pad(edge_index[0], (0, EP - E)).reshape(32, CPW, CH)
    dst_row = jnp.pad(edge_index[1], (0, EP - E)).reshape(1, EP)
    ea_p = jnp.pad(edge_attr, ((0, EP - E), (0, 0)))
    x_p = _pad2(x, NP, 128)
    batch_row = jnp.pad(batch, (0, NP - N), constant_values=-1).reshape(1, NP)

    w1flat = jnp.pad(W1b.reshape(32, IN, H1),
                     ((0, 0), (0, 0), (0, O1 - H1))).reshape(32 * IN, O1)
    b1mat = _pad2(b1b.reshape(IN, H1), IN, O1)
    w2flat = jnp.pad(W2b.reshape(32, H1, H2),
                     ((0, 0), (0, 128 - H1), (0, O2 - H2))).reshape(32 * 128, O2)
    b2mat = _pad2(b2b.reshape(H1, H2), 128, O2)

    root1p = _pad2(root1, 32, O1)
    root2p = _pad2(root2, 128, O2)
    bias1p = jnp.pad(bias1, (0, O1 - H1)).reshape(1, O1)
    gam1p = jnp.pad(gamma1, (0, O1 - H1)).reshape(1, O1)
    bet1p = jnp.pad(beta1, (0, O1 - H1)).reshape(1, O1)
    bias2p = jnp.pad(bias2, (0, O2 - H2)).reshape(1, O2)
    gam2p = jnp.pad(gamma2, (0, O2 - H2)).reshape(1, O2)
    bet2p = jnp.pad(beta2, (0, O2 - H2)).reshape(1, O2)
    w3p = _pad2(W3, O2, 128)
    b3p = jnp.pad(b3, (0, 128 - 64)).reshape(1, 128)
    w4p = _pad2(W4, 128, 128)
    b4b = jnp.broadcast_to(b4.reshape(1, 1), (1, 128))
    ba1 = b1a.reshape(1, 32)
    ba2 = b2a.reshape(1, 32)

    xs = _sc_gather(x_p, src2, 128)
    h1bn = _tc_layer1(dst_row, ea_p, xs, W1a, ba1, w1flat, b1mat,
                      x_p, root1p, bias1p, gam1p, bet1p)

    hs = _sc_gather(h1bn, src2, 128)
    out = _tc_layer2(dst_row, ea_p, hs, W2a, ba2, w2flat, b2mat,
                     h1bn, root2p, bias2p, gam2p, bet2p, batch_row,
                     w3p, b3p, w4p, b4b)
    return out[:, 0]
